# baseline (speedup 1.0000x reference)
import functools

import jax
import jax.numpy as jnp
from jax import lax
from jax.experimental import pallas as pl
from jax.experimental.pallas import tpu as pltpu
from jax.experimental.pallas import tpu_sc as plsc

VOCAB = 1000
D = 1000

NC = 2
NS = 16
NW = NC * NS
L = 16

DC = 40
DCP = 48
SKW = 64
NCHUNK = D // DC
NG = DC // 8


def _make_gather(B: int, S: int):
    assert B % (NW * 128) == 0 and B // 128 == NW
    n_units = S * NCHUNK

    mesh = plsc.VectorSubcoreMesh(core_axis_name="c", subcore_axis_name="s")

    @functools.partial(
        pl.kernel,
        mesh=mesh,
        out_type=jax.ShapeDtypeStruct((S, D // 8, NW, 8, 128), jnp.float32),
        scratch_types=[
            pltpu.VMEM((S, 128), jnp.int32),
            pltpu.VMEM((128,), jnp.int32),
            pltpu.VMEM((128,), jnp.int32),
            pltpu.VMEM((128, DCP), jnp.float32),
            pltpu.VMEM((128, DCP), jnp.float32),
            pltpu.VMEM((128 * SKW,), jnp.float32),
            pltpu.VMEM((NG, 8, 128), jnp.float32),
            pltpu.VMEM((NG, 8, 128), jnp.float32),
            pltpu.SemaphoreType.DMA,
            pltpu.SemaphoreType.DMA,
            pltpu.SemaphoreType.DMA,
            pltpu.SemaphoreType.DMA,
        ],
        compiler_params=pltpu.CompilerParams(
            use_tc_tiling_on_sc=False, needs_layout_passes=False),
    )
    def gather(idx_hbm, table_hbm, out_hbm, idxw, ind0, ind1, in0, in1,
               skb, t0, t1, g0, g1, o0, o1):
        w = lax.axis_index("s") * NC + lax.axis_index("c")
        pltpu.sync_copy(idx_hbm.at[w], idxw)

        iota = lax.iota(jnp.int32, L)
        rows = [iota + (br0 * L) for br0 in range(8)]

        def fill_ind(ind_v, i):
            s, c = i // NCHUNK, i % NCHUNK
            for br0 in range(8):
                v = idxw[s, pl.ds(br0 * L, L)]
                ind_v[pl.ds(br0 * L, L)] = v * NCHUNK + c

        def gstart(ind_v, buf, gsem, i):
            fill_ind(ind_v, i)
            pltpu.async_copy(table_hbm.at[ind_v], buf, gsem)

        def gwait(ind_v, buf, gsem):
            pltpu.make_async_copy(table_hbm.at[ind_v], buf, gsem).wait()

        def oview(i):
            s, c = i // NCHUNK, i % NCHUNK
            return out_hbm.at[s, pl.ds(c * NG, NG), w]

        def ostart(tbuf, osem, i):
            pltpu.async_copy(tbuf, oview(i), osem)

        def owait(tbuf, osem, i):
            pltpu.make_async_copy(tbuf, oview(i), osem).wait()

        cvecs = [iota * (SKW + 1) + (br0 * L * SKW) for br0 in range(8)]

        def transpose(buf, tbuf):
            def p1(jj, carry):
                for u in range(4):
                    br = 4 * jj + u
                    o = br * SKW + lax.rem(br, 16)
                    for kk in range(DCP // L):
                        skb[pl.ds(o + kk * L, L)] = buf[br, pl.ds(kk * L, L)]
                return carry
            lax.fori_loop(0, 32, p1, 0)

            def p2(d, carry):
                dg, dr = d // 8, d % 8
                ds_ = jnp.full((L,), d, jnp.int32)
                for br0 in range(8):
                    vals = plsc.load_gather(skb, [cvecs[br0] + ds_])
                    tbuf[dg, dr, pl.ds(br0 * L, L)] = vals
                return carry
            lax.fori_loop(0, DC, p2, 0)

        gstart(ind0, in0, g0, 0)

        def body(k, carry):
            i0 = 2 * k
            gwait(ind0, in0, g0)
            gstart(ind1, in1, g1, i0 + 1)

            @pl.when(i0 >= 2)
            def _():
                owait(t0, o0, i0 - 2)

            transpose(in0, t0)
            ostart(t0, o0, i0)
            gwait(ind1, in1, g1)

            @pl.when(i0 + 2 < n_units)
            def _():
                gstart(ind0, in0, g0, i0 + 2)

            @pl.when(i0 >= 1)
            def _():
                owait(t1, o1, i0 - 1)

            transpose(in1, t1)
            ostart(t1, o1, i0 + 1)
            return carry

        lax.fori_loop(0, n_units // 2, body, 0)
        owait(t0, o0, n_units - 2)
        owait(t1, o1, n_units - 1)

    return gather


def kernel(idx, token_table):
    b, s = idx.shape
    idx3 = jnp.transpose(idx.astype(jnp.int32).reshape(NW, 128, s), (0, 2, 1))
    table_p = jnp.pad(token_table.reshape(VOCAB * NCHUNK, DC),
                      ((0, 0), (0, DCP - DC)))
    out5 = _make_gather(b, s)(idx3, table_p)
    return jnp.transpose(out5, (2, 4, 0, 1, 3)).reshape(b, s, D)


# speedup vs baseline: 4.8820x; 4.8820x over previous
import functools

import jax
import jax.numpy as jnp
from jax import lax
from jax.experimental import pallas as pl
from jax.experimental.pallas import tpu as pltpu
from jax.experimental.pallas import tpu_sc as plsc

VOCAB = 1000
D = 1000

NC = 2
NS = 16
NW = NC * NS
L = 16

DC = 200
DCP = 208
NCHUNK = D // DC
NG = DC // 8


def _make_gather(B: int, S: int):
    assert B % (NW * 128) == 0 and B // 128 == NW
    n_units = S * NCHUNK

    mesh = plsc.VectorSubcoreMesh(core_axis_name="c", subcore_axis_name="s")

    @functools.partial(
        pl.kernel,
        mesh=mesh,
        out_type=jax.ShapeDtypeStruct((S, D // 8, NW, 8, 128), jnp.float32),
        scratch_types=[
            pltpu.VMEM((S, 128), jnp.int32),
            pltpu.VMEM((128,), jnp.int32),
            pltpu.VMEM((128,), jnp.int32),
            pltpu.VMEM((128, DCP), jnp.float32),
            pltpu.VMEM((128, DCP), jnp.float32),
            pltpu.VMEM((NG, 8, 128), jnp.float32),
            pltpu.VMEM((NG, 8, 128), jnp.float32),
            pltpu.SemaphoreType.DMA,
            pltpu.SemaphoreType.DMA,
            pltpu.SemaphoreType.DMA,
            pltpu.SemaphoreType.DMA,
        ],
        compiler_params=pltpu.CompilerParams(
            use_tc_tiling_on_sc=False, needs_layout_passes=False),
    )
    def gather(idx_hbm, table_hbm, out_hbm, idxw, ind0, ind1, in0, in1,
               t0, t1, g0, g1, o0, o1):
        w = lax.axis_index("s") * NC + lax.axis_index("c")
        pltpu.sync_copy(idx_hbm.at[w], idxw)

        iota = lax.iota(jnp.int32, L)
        rows = [iota + (br0 * L) for br0 in range(8)]

        def fill_ind(ind_v, i):
            s, c = i // NCHUNK, i % NCHUNK
            for br0 in range(8):
                v = idxw[s, pl.ds(br0 * L, L)]
                ind_v[pl.ds(br0 * L, L)] = v * NCHUNK + c

        def gstart(ind_v, buf, gsem, i):
            fill_ind(ind_v, i)
            pltpu.async_copy(table_hbm.at[ind_v], buf, gsem)

        def gwait(ind_v, buf, gsem):
            pltpu.make_async_copy(table_hbm.at[ind_v], buf, gsem).wait()

        def oview(i):
            s, c = i // NCHUNK, i % NCHUNK
            return out_hbm.at[s, pl.ds(c * NG, NG), w]

        def ostart(tbuf, osem, i):
            pltpu.async_copy(tbuf, oview(i), osem)

        def owait(tbuf, osem, i):
            pltpu.make_async_copy(tbuf, oview(i), osem).wait()

        def transpose(buf, tbuf):
            @functools.partial(plsc.parallel_loop, 0, NG, unroll=2)
            def _p(dg):
                d0 = dg * 8
                for dr in range(8):
                    col = jnp.full((L,), d0 + dr, jnp.int32)
                    for br0 in range(8):
                        vals = plsc.load_gather(buf, [rows[br0], col])
                        tbuf[dg, dr, pl.ds(br0 * L, L)] = vals

        gstart(ind0, in0, g0, 0)

        def body(k, carry):
            i0 = 2 * k
            gwait(ind0, in0, g0)
            gstart(ind1, in1, g1, i0 + 1)

            @pl.when(i0 >= 2)
            def _():
                owait(t0, o0, i0 - 2)

            transpose(in0, t0)
            ostart(t0, o0, i0)
            gwait(ind1, in1, g1)

            @pl.when(i0 + 2 < n_units)
            def _():
                gstart(ind0, in0, g0, i0 + 2)

            @pl.when(i0 >= 1)
            def _():
                owait(t1, o1, i0 - 1)

            transpose(in1, t1)
            ostart(t1, o1, i0 + 1)
            return carry

        lax.fori_loop(0, n_units // 2, body, 0)
        owait(t0, o0, n_units - 2)
        owait(t1, o1, n_units - 1)

    return gather


def kernel(idx, token_table):
    b, s = idx.shape
    idx3 = jnp.transpose(idx.astype(jnp.int32).reshape(NW, 128, s), (0, 2, 1))
    table_p = jnp.pad(token_table.reshape(VOCAB * NCHUNK, DC),
                      ((0, 0), (0, DCP - DC)))
    out5 = _make_gather(b, s)(idx3, table_p)
    return jnp.transpose(out5, (2, 4, 0, 1, 3)).reshape(b, s, D)


# speedup vs baseline: 5.2976x; 1.0851x over previous
import functools

import jax
import jax.numpy as jnp
from jax import lax
from jax.experimental import pallas as pl
from jax.experimental.pallas import tpu as pltpu
from jax.experimental.pallas import tpu_sc as plsc

VOCAB = 1000
D = 1000

NC = 2
NS = 16
NW = NC * NS
L = 16

DC = 200
DCP = 208
NCHUNK = D // DC
NG = DC // 8


def _make_gather(B: int, S: int):
    assert B % (NW * 128) == 0 and B // 128 == NW
    n_units = S * NCHUNK

    mesh = plsc.VectorSubcoreMesh(core_axis_name="c", subcore_axis_name="s")

    @functools.partial(
        pl.kernel,
        mesh=mesh,
        out_type=jax.ShapeDtypeStruct((S, D // 8, NW, 8, 128), jnp.float32),
        scratch_types=[
            pltpu.VMEM((S, 128), jnp.int32),
            pltpu.VMEM((128,), jnp.int32),
            pltpu.VMEM((128,), jnp.int32),
            pltpu.VMEM((128, DCP), jnp.float32),
            pltpu.VMEM((128, DCP), jnp.float32),
            pltpu.VMEM((NG, 8, 128), jnp.float32),
            pltpu.VMEM((NG, 8, 128), jnp.float32),
            pltpu.SemaphoreType.DMA,
            pltpu.SemaphoreType.DMA,
            pltpu.SemaphoreType.DMA,
            pltpu.SemaphoreType.DMA,
        ],
        compiler_params=pltpu.CompilerParams(
            use_tc_tiling_on_sc=False, needs_layout_passes=False),
    )
    def gather(idx_hbm, table_hbm, out_hbm, idxw, ind0, ind1, in0, in1,
               t0, t1, g0, g1, o0, o1):
        w = lax.axis_index("s") * NC + lax.axis_index("c")
        pltpu.sync_copy(idx_hbm.at[w], idxw)

        iota = lax.iota(jnp.int32, L)
        rows = [iota + (br0 * L) for br0 in range(8)]

        def fill_ind(ind_v, i):
            s, c = i // NCHUNK, i % NCHUNK
            for br0 in range(8):
                v = idxw[s, pl.ds(br0 * L, L)]
                ind_v[pl.ds(br0 * L, L)] = v * NCHUNK + c

        def gstart(ind_v, buf, gsem, i):
            fill_ind(ind_v, i)
            pltpu.async_copy(table_hbm.at[ind_v], buf, gsem)

        def gwait(ind_v, buf, gsem):
            pltpu.make_async_copy(table_hbm.at[ind_v], buf, gsem).wait()

        def oview(i):
            s, c = i // NCHUNK, i % NCHUNK
            return out_hbm.at[s, pl.ds(c * NG, NG), w]

        def ostart(tbuf, osem, i):
            pltpu.async_copy(tbuf, oview(i), osem)

        def owait(tbuf, osem, i):
            pltpu.make_async_copy(tbuf, oview(i), osem).wait()

        def transpose(buf, tbuf):
            @functools.partial(plsc.parallel_loop, 0, NG, unroll=2)
            def _p(dg):
                d0 = dg * 8
                for dr in range(8):
                    col = jnp.full((L,), d0 + dr, jnp.int32)
                    for br0 in range(8):
                        vals = plsc.load_gather(buf, [rows[br0], col])
                        tbuf[dg, dr, pl.ds(br0 * L, L)] = vals

        gstart(ind0, in0, g0, 0)

        def body(k, carry):
            i0 = 2 * k
            gstart(ind1, in1, g1, i0 + 1)
            gwait(ind0, in0, g0)

            @pl.when(i0 >= 2)
            def _():
                owait(t0, o0, i0 - 2)

            transpose(in0, t0)
            ostart(t0, o0, i0)
            gwait(ind1, in1, g1)

            @pl.when(i0 + 2 < n_units)
            def _():
                gstart(ind0, in0, g0, i0 + 2)

            @pl.when(i0 >= 1)
            def _():
                owait(t1, o1, i0 - 1)

            transpose(in1, t1)
            ostart(t1, o1, i0 + 1)
            return carry

        lax.fori_loop(0, n_units // 2, body, 0)
        owait(t0, o0, n_units - 2)
        owait(t1, o1, n_units - 1)

    return gather


def kernel(idx, token_table):
    b, s = idx.shape
    idx3 = jnp.transpose(idx.astype(jnp.int32).reshape(NW, 128, s), (0, 2, 1))
    table_p = jnp.pad(token_table.reshape(VOCAB * NCHUNK, DC),
                      ((0, 0), (0, DCP - DC)))
    out5 = _make_gather(b, s)(idx3, table_p)
    return jnp.transpose(out5, (2, 4, 0, 1, 3)).reshape(b, s, D)
